# initial kernel scaffold (unmeasured)
import functools

import jax
import jax.numpy as jnp
from jax import lax
from jax.experimental import pallas as pl
from jax.experimental.pallas import tpu as pltpu

B, S, H, Dh, Dr = 4, 256, 32, 128, 64


def _kv_exchange(x2, Wdkv, Wuk, Wuv):
    M, D = x2.shape
    dc = Wdkv.shape[1]
    N = Wuk.shape[1]

    def body(x_ref, wdkv_ref, wuk_ref, wuv_ref,
             c_mine_ref, c_peer_ref, wuk_peer_ref, wuv_peer_ref,
             send_sems, recv_sems):
        my_x = lax.axis_index("x")
        my_y = lax.axis_index("y")
        my_z = lax.axis_index("z")
        peer = (1 - my_x, my_y, my_z)

        barrier = pltpu.get_barrier_semaphore()
        pl.semaphore_signal(barrier, inc=1, device_id=peer,
                            device_id_type=pl.DeviceIdType.MESH)
        pl.semaphore_wait(barrier, 1)

        c_mine_ref[...] = jnp.dot(x_ref[...], wdkv_ref[...],
                                  preferred_element_type=jnp.float32)

        rdma_c = pltpu.make_async_remote_copy(
            src_ref=c_mine_ref, dst_ref=c_peer_ref,
            send_sem=send_sems.at[0], recv_sem=recv_sems.at[0],
            device_id=peer, device_id_type=pl.DeviceIdType.MESH)
        rdma_uk = pltpu.make_async_remote_copy(
            src_ref=wuk_ref, dst_ref=wuk_peer_ref,
            send_sem=send_sems.at[1], recv_sem=recv_sems.at[1],
            device_id=peer, device_id_type=pl.DeviceIdType.MESH)
        rdma_uv = pltpu.make_async_remote_copy(
            src_ref=wuv_ref, dst_ref=wuv_peer_ref,
            send_sem=send_sems.at[2], recv_sem=recv_sems.at[2],
            device_id=peer, device_id_type=pl.DeviceIdType.MESH)
        rdma_c.start()
        rdma_uk.start()
        rdma_uv.start()
        rdma_c.wait()
        rdma_uk.wait()
        rdma_uv.wait()

    vmem = pl.BlockSpec(memory_space=pltpu.VMEM)
    return pl.pallas_call(
        body,
        out_shape=(
            jax.ShapeDtypeStruct((M, dc), jnp.float32),
            jax.ShapeDtypeStruct((M, dc), jnp.float32),
            jax.ShapeDtypeStruct((dc, N), jnp.float32),
            jax.ShapeDtypeStruct((dc, N), jnp.float32),
        ),
        in_specs=[vmem, vmem, vmem, vmem],
        out_specs=(vmem, vmem, vmem, vmem),
        scratch_shapes=[
            pltpu.SemaphoreType.DMA((3,)),
            pltpu.SemaphoreType.DMA((3,)),
        ],
        compiler_params=pltpu.CompilerParams(collective_id=0),
    )(x2, Wdkv, Wuk, Wuv)


def _kv_combine(c_mine, c_peer, Wuk_m, Wuk_p, Wuv_m, Wuv_p, block_n=512):
    M, dc = c_mine.shape
    N = Wuk_m.shape[1]

    def body(cm_ref, cp_ref, ukm_ref, ukp_ref, uvm_ref, uvp_ref,
             k_ref, v_ref):
        cm = cm_ref[...]
        cp = cp_ref[...]
        k_ref[...] = (jnp.dot(cm, ukm_ref[...], preferred_element_type=jnp.float32)
                      + jnp.dot(cp, ukp_ref[...], preferred_element_type=jnp.float32))
        v_ref[...] = (jnp.dot(cm, uvm_ref[...], preferred_element_type=jnp.float32)
                      + jnp.dot(cp, uvp_ref[...], preferred_element_type=jnp.float32))

    full_c = pl.BlockSpec((M, dc), lambda j: (0, 0))
    wblk = pl.BlockSpec((dc, block_n), lambda j: (0, j))
    oblk = pl.BlockSpec((M, block_n), lambda j: (0, j))
    return pl.pallas_call(
        body,
        grid=(N // block_n,),
        in_specs=[full_c, full_c, wblk, wblk, wblk, wblk],
        out_specs=(oblk, oblk),
        out_shape=(
            jax.ShapeDtypeStruct((M, N), jnp.float32),
            jax.ShapeDtypeStruct((M, N), jnp.float32),
        ),
    )(c_mine, c_peer, Wuk_m, Wuk_p, Wuv_m, Wuv_p)


def _matmul(a, w, block_n=512):
    M, K = a.shape
    _, N = w.shape
    block_n = min(block_n, N)

    def body(a_ref, w_ref, o_ref):
        o_ref[...] = jnp.dot(a_ref[...], w_ref[...],
                             preferred_element_type=jnp.float32)

    return pl.pallas_call(
        body,
        grid=(N // block_n,),
        in_specs=[
            pl.BlockSpec((M, K), lambda j: (0, 0)),
            pl.BlockSpec((K, block_n), lambda j: (0, j)),
        ],
        out_specs=pl.BlockSpec((M, block_n), lambda j: (0, j)),
        out_shape=jax.ShapeDtypeStruct((M, N), jnp.float32),
    )(a, w)


def _attention(Q4, K4, V4, Qr4, Kr3):
    scale = (Dh + Dr) ** -0.5

    def body(q_ref, k_ref, v_ref, qr_ref, kr_ref, o_ref):
        q = q_ref[0, :, 0, :]
        k = k_ref[0, :, 0, :]
        v = v_ref[0, :, 0, :]
        qr = qr_ref[0, :, 0, :]
        kr = kr_ref[0, :, :]
        dot_t = lambda a, b: lax.dot_general(
            a, b, (((1,), (1,)), ((), ())), preferred_element_type=jnp.float32)
        s = (dot_t(q, k) + dot_t(qr, kr)) * scale
        m = jnp.max(s, axis=-1, keepdims=True)
        p = jnp.exp(s - m)
        p = p / jnp.sum(p, axis=-1, keepdims=True)
        o_ref[0, :, 0, :] = jnp.dot(p, v, preferred_element_type=jnp.float32)

    qspec = pl.BlockSpec((1, S, 1, Dh), lambda b, h: (b, 0, h, 0))
    rspec = pl.BlockSpec((1, S, 1, Dr), lambda b, h: (b, 0, h, 0))
    krspec = pl.BlockSpec((1, S, Dr), lambda b, h: (b, 0, 0))
    return pl.pallas_call(
        body,
        grid=(B, H),
        in_specs=[qspec, qspec, qspec, rspec, krspec],
        out_specs=qspec,
        out_shape=jax.ShapeDtypeStruct((B, S, H, Dh), jnp.float32),
    )(Q4, K4, V4, Qr4, Kr3)


def kernel(x, Wdkv, Wuk, Wuv, Wq, Wqr, Wkr, Wo):
    x2 = x.reshape(B * S, -1)

    c_mine, c_peer, Wuk_p, Wuv_p = _kv_exchange(x2, Wdkv, Wuk, Wuv)
    K2, V2 = _kv_combine(c_mine, c_peer, Wuk, Wuk_p, Wuv, Wuv_p)

    Q2 = _matmul(x2, Wq)
    Qr2 = _matmul(x2, Wqr)
    Kr2 = _matmul(x2, Wkr)

    Q4 = Q2.reshape(B, S, H, Dh)
    K4 = K2.reshape(B, S, H, Dh)
    V4 = V2.reshape(B, S, H, Dh)
    Qr4 = Qr2.reshape(B, S, H, Dr)
    Kr3 = Kr2.reshape(B, S, Dr)

    O4 = _attention(Q4, K4, V4, Qr4, Kr3)
    O2 = O4.reshape(B * S, H * Dh)

    out = _matmul(O2, Wo)
    return out.reshape(B, S, H * Dh)


# baseline (device time: 304186 ns/iter reference)
import functools

import jax
import jax.numpy as jnp
from jax import lax
from jax.experimental import pallas as pl
from jax.experimental.pallas import tpu as pltpu

B, S, H, Dh, Dr = 4, 256, 32, 128, 64
_VMEM_LIMIT = 100 * 1024 * 1024
_CP = pltpu.CompilerParams(vmem_limit_bytes=_VMEM_LIMIT)


def _kv_exchange(x2, Wdkv, Wuk, Wuv):
    M, D = x2.shape
    dc = Wdkv.shape[1]
    N = Wuk.shape[1]

    def body(x_ref, wdkv_ref, wuk_ref, wuv_ref,
             c_mine_ref, c_peer_ref, wuk_peer_ref, wuv_peer_ref,
             send_sems, recv_sems):
        my_x = lax.axis_index("x")
        my_y = lax.axis_index("y")
        my_z = lax.axis_index("z")
        peer = (1 - my_x, my_y, my_z)

        barrier = pltpu.get_barrier_semaphore()
        pl.semaphore_signal(barrier, inc=1, device_id=peer,
                            device_id_type=pl.DeviceIdType.MESH)
        pl.semaphore_wait(barrier, 1)

        c_mine_ref[...] = jnp.dot(x_ref[...], wdkv_ref[...],
                                  preferred_element_type=jnp.float32)

        rdma_c = pltpu.make_async_remote_copy(
            src_ref=c_mine_ref, dst_ref=c_peer_ref,
            send_sem=send_sems.at[0], recv_sem=recv_sems.at[0],
            device_id=peer, device_id_type=pl.DeviceIdType.MESH)
        rdma_uk = pltpu.make_async_remote_copy(
            src_ref=wuk_ref, dst_ref=wuk_peer_ref,
            send_sem=send_sems.at[1], recv_sem=recv_sems.at[1],
            device_id=peer, device_id_type=pl.DeviceIdType.MESH)
        rdma_uv = pltpu.make_async_remote_copy(
            src_ref=wuv_ref, dst_ref=wuv_peer_ref,
            send_sem=send_sems.at[2], recv_sem=recv_sems.at[2],
            device_id=peer, device_id_type=pl.DeviceIdType.MESH)
        rdma_c.start()
        rdma_uk.start()
        rdma_uv.start()
        rdma_c.wait()
        rdma_uk.wait()
        rdma_uv.wait()

    vmem = pl.BlockSpec(memory_space=pltpu.VMEM)
    return pl.pallas_call(
        body,
        out_shape=(
            jax.ShapeDtypeStruct((M, dc), jnp.float32),
            jax.ShapeDtypeStruct((M, dc), jnp.float32),
            jax.ShapeDtypeStruct((dc, N), jnp.float32),
            jax.ShapeDtypeStruct((dc, N), jnp.float32),
        ),
        in_specs=[vmem, vmem, vmem, vmem],
        out_specs=(vmem, vmem, vmem, vmem),
        scratch_shapes=[
            pltpu.SemaphoreType.DMA((3,)),
            pltpu.SemaphoreType.DMA((3,)),
        ],
        compiler_params=pltpu.CompilerParams(collective_id=0, vmem_limit_bytes=_VMEM_LIMIT),
    )(x2, Wdkv, Wuk, Wuv)


def _kv_combine(c_mine, c_peer, Wuk_m, Wuk_p, Wuv_m, Wuv_p, block_n=512):
    M, dc = c_mine.shape
    N = Wuk_m.shape[1]

    def body(cm_ref, cp_ref, ukm_ref, ukp_ref, uvm_ref, uvp_ref,
             k_ref, v_ref):
        cm = cm_ref[...]
        cp = cp_ref[...]
        k_ref[...] = (jnp.dot(cm, ukm_ref[...], preferred_element_type=jnp.float32)
                      + jnp.dot(cp, ukp_ref[...], preferred_element_type=jnp.float32))
        v_ref[...] = (jnp.dot(cm, uvm_ref[...], preferred_element_type=jnp.float32)
                      + jnp.dot(cp, uvp_ref[...], preferred_element_type=jnp.float32))

    full_c = pl.BlockSpec((M, dc), lambda j: (0, 0))
    wblk = pl.BlockSpec((dc, block_n), lambda j: (0, j))
    oblk = pl.BlockSpec((M, block_n), lambda j: (0, j))
    return pl.pallas_call(
        body,
        grid=(N // block_n,),
        in_specs=[full_c, full_c, wblk, wblk, wblk, wblk],
        out_specs=(oblk, oblk),
        out_shape=(
            jax.ShapeDtypeStruct((M, N), jnp.float32),
            jax.ShapeDtypeStruct((M, N), jnp.float32),
        ),
        compiler_params=_CP,
    )(c_mine, c_peer, Wuk_m, Wuk_p, Wuv_m, Wuv_p)


def _matmul(a, w, block_n=512):
    M, K = a.shape
    _, N = w.shape
    block_n = min(block_n, N)

    def body(a_ref, w_ref, o_ref):
        o_ref[...] = jnp.dot(a_ref[...], w_ref[...],
                             preferred_element_type=jnp.float32)

    return pl.pallas_call(
        body,
        grid=(N // block_n,),
        in_specs=[
            pl.BlockSpec((M, K), lambda j: (0, 0)),
            pl.BlockSpec((K, block_n), lambda j: (0, j)),
        ],
        out_specs=pl.BlockSpec((M, block_n), lambda j: (0, j)),
        out_shape=jax.ShapeDtypeStruct((M, N), jnp.float32),
        compiler_params=_CP,
    )(a, w)


def _attention(Q3, K3, V3, Qr3, Kr3):
    scale = (Dh + Dr) ** -0.5

    def body(q_ref, k_ref, v_ref, qr_ref, kr_ref, o_ref):
        kr = kr_ref[0]
        dot_t = lambda a, b: lax.dot_general(
            a, b, (((1,), (1,)), ((), ())), preferred_element_type=jnp.float32)
        for i in range(2):
            q = q_ref[0, :, i * Dh:(i + 1) * Dh]
            k = k_ref[0, :, i * Dh:(i + 1) * Dh]
            v = v_ref[0, :, i * Dh:(i + 1) * Dh]
            qr = qr_ref[0, :, i * Dr:(i + 1) * Dr]
            s = (dot_t(q, k) + dot_t(qr, kr)) * scale
            m = jnp.max(s, axis=-1, keepdims=True)
            p = jnp.exp(s - m)
            p = p / jnp.sum(p, axis=-1, keepdims=True)
            o_ref[0, :, i * Dh:(i + 1) * Dh] = jnp.dot(
                p, v, preferred_element_type=jnp.float32)

    qspec = pl.BlockSpec((1, S, 2 * Dh), lambda b, h: (b, 0, h))
    rspec = pl.BlockSpec((1, S, 2 * Dr), lambda b, h: (b, 0, h))
    krspec = pl.BlockSpec((1, S, Dr), lambda b, h: (b, 0, 0))
    return pl.pallas_call(
        body,
        grid=(B, H // 2),
        in_specs=[qspec, qspec, qspec, rspec, krspec],
        out_specs=qspec,
        out_shape=jax.ShapeDtypeStruct((B, S, H * Dh), jnp.float32),
        compiler_params=_CP,
    )(Q3, K3, V3, Qr3, Kr3)


def kernel(x, Wdkv, Wuk, Wuv, Wq, Wqr, Wkr, Wo):
    x2 = x.reshape(B * S, -1)

    c_mine, c_peer, Wuk_p, Wuv_p = _kv_exchange(x2, Wdkv, Wuk, Wuv)
    K2, V2 = _kv_combine(c_mine, c_peer, Wuk, Wuk_p, Wuv, Wuv_p)

    Q2 = _matmul(x2, Wq)
    Qr2 = _matmul(x2, Wqr)
    Kr2 = _matmul(x2, Wkr)

    Q3 = Q2.reshape(B, S, H * Dh)
    K3 = K2.reshape(B, S, H * Dh)
    V3 = V2.reshape(B, S, H * Dh)
    Qr3 = Qr2.reshape(B, S, H * Dr)
    Kr3 = Kr2.reshape(B, S, Dr)

    O3 = _attention(Q3, K3, V3, Qr3, Kr3)
    O2 = O3.reshape(B * S, H * Dh)

    out = _matmul(O2, Wo)
    return out.reshape(B, S, H * Dh)


# device time: 198292 ns/iter; 1.5340x vs baseline; 1.5340x over previous
import jax
import jax.numpy as jnp
from jax import lax
from jax.experimental import pallas as pl
from jax.experimental.pallas import tpu as pltpu

B, S, H, Dh, Dr = 4, 256, 32, 128, 64
D = H * Dh
MB = 128
_VMEM_LIMIT = 100 * 1024 * 1024
_CP = pltpu.CompilerParams(vmem_limit_bytes=_VMEM_LIMIT)
_MESH = pl.DeviceIdType.MESH

NQ_BLK = 16
NQR_BLK = 8
GRID_A = NQ_BLK + NQR_BLK


def _proj_exchange(x_batch, Wdkv, Wuk, Wuv, Wkr, Wq, Wqr):
    dc = Wdkv.shape[1]

    def body(xb_ref, wdkv_ref, wuk_ref, wuv_ref, wkr_ref, wq_ref, wqr_ref,
             q_ref, qr_ref, kr_ref, k_ref, v_ref,
             c_mine, c_peer, wuk_p, wuv_p, send_sems, recv_sems):
        j = pl.program_id(0)
        my_x = lax.axis_index("x")
        peer = (1 - my_x, lax.axis_index("y"), lax.axis_index("z"))

        rdma_c = pltpu.make_async_remote_copy(
            src_ref=c_mine, dst_ref=c_peer,
            send_sem=send_sems.at[0], recv_sem=recv_sems.at[0],
            device_id=peer, device_id_type=_MESH)
        rdma_uk = pltpu.make_async_remote_copy(
            src_ref=wuk_ref, dst_ref=wuk_p,
            send_sem=send_sems.at[1], recv_sem=recv_sems.at[1],
            device_id=peer, device_id_type=_MESH)
        rdma_uv = pltpu.make_async_remote_copy(
            src_ref=wuv_ref, dst_ref=wuv_p,
            send_sem=send_sems.at[2], recv_sem=recv_sems.at[2],
            device_id=peer, device_id_type=_MESH)

        @pl.when(j == 0)
        def _start():
            barrier = pltpu.get_barrier_semaphore()
            pl.semaphore_signal(barrier, inc=1, device_id=peer,
                                device_id_type=_MESH)
            pl.semaphore_wait(barrier, 1)
            c_mine[...] = jnp.dot(xb_ref[...], wdkv_ref[...],
                                  preferred_element_type=jnp.float32)
            rdma_c.start()
            rdma_uk.start()
            rdma_uv.start()
            kr_ref[...] = jnp.dot(xb_ref[...], wkr_ref[...],
                                  preferred_element_type=jnp.float32)

        x_mine = xb_ref[pl.ds(my_x * MB, MB), :]

        @pl.when(j < NQ_BLK)
        def _q():
            q_ref[...] = jnp.dot(x_mine, wq_ref[...],
                                 preferred_element_type=jnp.float32)

        @pl.when(j >= NQ_BLK)
        def _qr():
            qr_ref[...] = jnp.dot(x_mine, wqr_ref[...],
                                  preferred_element_type=jnp.float32)

        @pl.when(j == GRID_A - 1)
        def _finish():
            rdma_c.wait()
            rdma_uk.wait()
            rdma_uv.wait()
            cm = c_mine[...]
            cp = c_peer[...]
            k_ref[...] = (
                jnp.dot(cm, wuk_ref[...], preferred_element_type=jnp.float32)
                + jnp.dot(cp, wuk_p[...], preferred_element_type=jnp.float32))
            v_ref[...] = (
                jnp.dot(cm, wuv_ref[...], preferred_element_type=jnp.float32)
                + jnp.dot(cp, wuv_p[...], preferred_element_type=jnp.float32))

    full = lambda shape: pl.BlockSpec(shape, lambda j: (0,) * len(shape))
    wq_spec = pl.BlockSpec((D, 256), lambda j: (0, jnp.minimum(j, NQ_BLK - 1)))
    wqr_spec = pl.BlockSpec(
        (D, 256), lambda j: (0, jnp.clip(j - NQ_BLK, 0, NQR_BLK - 1)))

    return pl.pallas_call(
        body,
        grid=(GRID_A,),
        in_specs=[
            full((2 * MB, D)),
            full((D, dc)),
            full((dc, D)),
            full((dc, D)),
            full((D, Dr)),
            wq_spec,
            wqr_spec,
        ],
        out_specs=(
            pl.BlockSpec((MB, 256), lambda j: (0, jnp.minimum(j, NQ_BLK - 1))),
            pl.BlockSpec((MB, 256),
                         lambda j: (0, jnp.clip(j - NQ_BLK, 0, NQR_BLK - 1))),
            full((2 * MB, Dr)),
            full((2 * MB, D)),
            full((2 * MB, D)),
        ),
        out_shape=(
            jax.ShapeDtypeStruct((MB, D), jnp.float32),
            jax.ShapeDtypeStruct((MB, H * Dr), jnp.float32),
            jax.ShapeDtypeStruct((2 * MB, Dr), jnp.float32),
            jax.ShapeDtypeStruct((2 * MB, D), jnp.float32),
            jax.ShapeDtypeStruct((2 * MB, D), jnp.float32),
        ),
        scratch_shapes=[
            pltpu.VMEM((2 * MB, dc), jnp.float32),
            pltpu.VMEM((2 * MB, dc), jnp.float32),
            pltpu.VMEM((dc, D), jnp.float32),
            pltpu.VMEM((dc, D), jnp.float32),
            pltpu.SemaphoreType.DMA((3,)),
            pltpu.SemaphoreType.DMA((3,)),
        ],
        compiler_params=pltpu.CompilerParams(
            collective_id=0, vmem_limit_bytes=_VMEM_LIMIT,
            dimension_semantics=("arbitrary",)),
    )(x_batch, Wdkv, Wuk, Wuv, Wkr, Wq, Wqr)


def _attention(Q_m, Qr_m, Kr_b, K_b, V_b):
    scale = (Dh + Dr) ** -0.5

    def body(q_ref, qr_ref, kr_ref, k_ref, v_ref, o_ref):
        kr = kr_ref[...]
        dot_t = lambda a, b: lax.dot_general(
            a, b, (((1,), (1,)), ((), ())), preferred_element_type=jnp.float32)
        for i in range(2):
            q = q_ref[:, i * Dh:(i + 1) * Dh]
            k = k_ref[:, i * Dh:(i + 1) * Dh]
            v = v_ref[:, i * Dh:(i + 1) * Dh]
            qr = qr_ref[:, i * Dr:(i + 1) * Dr]
            s = (dot_t(q, k) + dot_t(qr, kr)) * scale
            m = jnp.max(s, axis=-1, keepdims=True)
            p = jnp.exp(s - m)
            p = p / jnp.sum(p, axis=-1, keepdims=True)
            o_ref[:, i * Dh:(i + 1) * Dh] = jnp.dot(
                p, v, preferred_element_type=jnp.float32)

    return pl.pallas_call(
        body,
        grid=(H // 2,),
        in_specs=[
            pl.BlockSpec((MB, 2 * Dh), lambda h: (0, h)),
            pl.BlockSpec((MB, 2 * Dr), lambda h: (0, h)),
            pl.BlockSpec((2 * MB, Dr), lambda h: (0, 0)),
            pl.BlockSpec((2 * MB, 2 * Dh), lambda h: (0, h)),
            pl.BlockSpec((2 * MB, 2 * Dh), lambda h: (0, h)),
        ],
        out_specs=pl.BlockSpec((MB, 2 * Dh), lambda h: (0, h)),
        out_shape=jax.ShapeDtypeStruct((MB, D), jnp.float32),
        compiler_params=_CP,
    )(Q_m, Qr_m, Kr_b, K_b, V_b)


def _matmul(a, w, block_n=512):
    M, K = a.shape
    _, N = w.shape
    block_n = min(block_n, N)

    def body(a_ref, w_ref, o_ref):
        o_ref[...] = jnp.dot(a_ref[...], w_ref[...],
                             preferred_element_type=jnp.float32)

    return pl.pallas_call(
        body,
        grid=(N // block_n,),
        in_specs=[
            pl.BlockSpec((M, K), lambda j: (0, 0)),
            pl.BlockSpec((K, block_n), lambda j: (0, j)),
        ],
        out_specs=pl.BlockSpec((M, block_n), lambda j: (0, j)),
        out_shape=jax.ShapeDtypeStruct((M, N), jnp.float32),
        compiler_params=_CP,
    )(a, w)


def _allgather_out(out_mine):
    SPLITS = ((0, 48), (48, 48), (96, 32))

    def body(in_ref, out_ref, send_sems, recv_sems):
        bx = lax.axis_index("x")
        by = lax.axis_index("y")
        bz = lax.axis_index("z")
        nbrs = [(1 - bx, by, bz), (bx, 1 - by, bz), (bx, by, 1 - bz)]

        def blk(cx, cy, cz):
            return ((cy * 2 + cz) * 2 + cx) * MB

        me = blk(bx, by, bz)
        x_o = blk(1 - bx, by, bz)
        y_o = blk(bx, 1 - by, bz)
        z_o = blk(bx, by, 1 - bz)
        xy_o = blk(1 - bx, 1 - by, bz)
        xz_o = blk(1 - bx, by, 1 - bz)
        yz_o = blk(bx, 1 - by, 1 - bz)
        anti_o = blk(1 - bx, 1 - by, 1 - bz)

        barrier = pltpu.get_barrier_semaphore()
        for n in nbrs:
            pl.semaphore_signal(barrier, inc=1, device_id=n,
                                device_id_type=_MESH)
        pl.semaphore_wait(barrier, 3)

        out_ref[pl.ds(me, MB), :] = in_ref[...]

        def xfer(link, step, src_off, n_rows):
            return pltpu.make_async_remote_copy(
                src_ref=out_ref.at[pl.ds(src_off, n_rows), :],
                dst_ref=out_ref.at[pl.ds(src_off, n_rows), :],
                send_sem=send_sems.at[link, step],
                recv_sem=recv_sems.at[link, step],
                device_id=nbrs[link], device_id_type=_MESH)

        s1 = [xfer(l, 0, me, MB) for l in range(3)]
        for r in s1:
            r.start()
        for r in s1:
            r.wait()
        s2 = [xfer(0, 1, y_o, MB), xfer(1, 1, z_o, MB), xfer(2, 1, x_o, MB)]
        for r in s2:
            r.start()
        for r in s2:
            r.wait()
        srcs = (yz_o, xz_o, xy_o)
        s3 = [xfer(l, 2, srcs[l] + SPLITS[l][0], SPLITS[l][1])
              for l in range(3)]
        for r in s3:
            r.start()
        for r in s3:
            r.wait()

    return pl.pallas_call(
        body,
        out_shape=jax.ShapeDtypeStruct((B * S, D), jnp.float32),
        in_specs=[pl.BlockSpec(memory_space=pltpu.VMEM)],
        out_specs=pl.BlockSpec(memory_space=pltpu.VMEM),
        scratch_shapes=[
            pltpu.SemaphoreType.DMA((3, 3)),
            pltpu.SemaphoreType.DMA((3, 3)),
        ],
        compiler_params=pltpu.CompilerParams(
            collective_id=1, vmem_limit_bytes=_VMEM_LIMIT),
    )(out_mine)


def kernel(x, Wdkv, Wuk, Wuv, Wq, Wqr, Wkr, Wo):
    x2 = x.reshape(B * S, D)
    b = lax.axis_index("y") * 2 + lax.axis_index("z")
    x_batch = lax.dynamic_slice(x2, (b * S, 0), (S, D))

    Q_m, Qr_m, Kr_b, K_b, V_b = _proj_exchange(
        x_batch, Wdkv, Wuk, Wuv, Wkr, Wq, Wqr)

    O_m = _attention(Q_m, Qr_m, Kr_b, K_b, V_b)
    out_mine = _matmul(O_m, Wo)

    out = _allgather_out(out_mine)
    return out.reshape(B, S, D)


# device time: 162175 ns/iter; 1.8757x vs baseline; 1.2227x over previous
import jax
import jax.numpy as jnp
from jax import lax
from jax.experimental import pallas as pl
from jax.experimental.pallas import tpu as pltpu

B, S, H, Dh, Dr = 4, 256, 32, 128, 64
D = H * Dh
MB = 128
_VMEM_LIMIT = 100 * 1024 * 1024
_CP = pltpu.CompilerParams(vmem_limit_bytes=_VMEM_LIMIT)
_MESH = pl.DeviceIdType.MESH

NQ_BLK = 16
NQR_BLK = 8
GRID_A = NQ_BLK + NQR_BLK


def _proj_exchange(x2, Wdkv, Wuk, Wuv, Wkr, Wq, Wqr):
    dc = Wdkv.shape[1]

    def body(x_ref, wdkv_ref, wuk_ref, wuv_ref, wkr_ref, wq_ref, wqr_ref,
             q_ref, qr_ref, kr_ref, k_ref, v_ref,
             c_mine, c_peer, wuk_bf, wuv_bf, wuk_p, wuv_p,
             send_sems, recv_sems):
        j = pl.program_id(0)
        my_x = lax.axis_index("x")
        b = lax.axis_index("y") * 2 + lax.axis_index("z")
        peer = (1 - my_x, lax.axis_index("y"), lax.axis_index("z"))
        boff = b * S

        rdma_c = pltpu.make_async_remote_copy(
            src_ref=c_mine, dst_ref=c_peer,
            send_sem=send_sems.at[0], recv_sem=recv_sems.at[0],
            device_id=peer, device_id_type=_MESH)
        rdma_uk = pltpu.make_async_remote_copy(
            src_ref=wuk_bf, dst_ref=wuk_p,
            send_sem=send_sems.at[1], recv_sem=recv_sems.at[1],
            device_id=peer, device_id_type=_MESH)
        rdma_uv = pltpu.make_async_remote_copy(
            src_ref=wuv_bf, dst_ref=wuv_p,
            send_sem=send_sems.at[2], recv_sem=recv_sems.at[2],
            device_id=peer, device_id_type=_MESH)

        @pl.when(j == 0)
        def _start():
            barrier = pltpu.get_barrier_semaphore()
            pl.semaphore_signal(barrier, inc=1, device_id=peer,
                                device_id_type=_MESH)
            pl.semaphore_wait(barrier, 1)
            xb = x_ref[pl.ds(boff, S), :]
            c_mine[...] = jnp.dot(
                xb, wdkv_ref[...],
                preferred_element_type=jnp.float32).astype(jnp.bfloat16)
            wuk_bf[...] = wuk_ref[...].astype(jnp.bfloat16)
            wuv_bf[...] = wuv_ref[...].astype(jnp.bfloat16)
            rdma_c.start()
            rdma_uk.start()
            rdma_uv.start()
            kr_ref[...] = jnp.dot(xb, wkr_ref[...],
                                  preferred_element_type=jnp.float32)

        x_mine = x_ref[pl.ds(boff + my_x * MB, MB), :]

        @pl.when(j < NQ_BLK)
        def _q():
            q_ref[...] = jnp.dot(x_mine, wq_ref[...],
                                 preferred_element_type=jnp.float32)

        @pl.when(j >= NQ_BLK)
        def _qr():
            qr_ref[...] = jnp.dot(x_mine, wqr_ref[...],
                                  preferred_element_type=jnp.float32)

        @pl.when(j == GRID_A - 1)
        def _finish():
            rdma_c.wait()
            rdma_uk.wait()
            rdma_uv.wait()
            cm = c_mine[...]
            cp = c_peer[...]
            k_ref[...] = (
                jnp.dot(cm, wuk_bf[...], preferred_element_type=jnp.float32)
                + jnp.dot(cp, wuk_p[...], preferred_element_type=jnp.float32))
            v_ref[...] = (
                jnp.dot(cm, wuv_bf[...], preferred_element_type=jnp.float32)
                + jnp.dot(cp, wuv_p[...], preferred_element_type=jnp.float32))

    full = lambda shape: pl.BlockSpec(shape, lambda j: (0,) * len(shape))
    wq_spec = pl.BlockSpec((D, 256), lambda j: (0, jnp.minimum(j, NQ_BLK - 1)))
    wqr_spec = pl.BlockSpec(
        (D, 256), lambda j: (0, jnp.clip(j - NQ_BLK, 0, NQR_BLK - 1)))

    return pl.pallas_call(
        body,
        grid=(GRID_A,),
        in_specs=[
            full((B * S, D)),
            full((D, dc)),
            full((dc, D)),
            full((dc, D)),
            full((D, Dr)),
            wq_spec,
            wqr_spec,
        ],
        out_specs=(
            pl.BlockSpec((MB, 256), lambda j: (0, jnp.minimum(j, NQ_BLK - 1))),
            pl.BlockSpec((MB, 256),
                         lambda j: (0, jnp.clip(j - NQ_BLK, 0, NQR_BLK - 1))),
            full((S, Dr)),
            full((S, D)),
            full((S, D)),
        ),
        out_shape=(
            jax.ShapeDtypeStruct((MB, D), jnp.float32),
            jax.ShapeDtypeStruct((MB, H * Dr), jnp.float32),
            jax.ShapeDtypeStruct((S, Dr), jnp.float32),
            jax.ShapeDtypeStruct((S, D), jnp.float32),
            jax.ShapeDtypeStruct((S, D), jnp.float32),
        ),
        scratch_shapes=[
            pltpu.VMEM((S, dc), jnp.bfloat16),
            pltpu.VMEM((S, dc), jnp.bfloat16),
            pltpu.VMEM((dc, D), jnp.bfloat16),
            pltpu.VMEM((dc, D), jnp.bfloat16),
            pltpu.VMEM((dc, D), jnp.bfloat16),
            pltpu.VMEM((dc, D), jnp.bfloat16),
            pltpu.SemaphoreType.DMA((3,)),
            pltpu.SemaphoreType.DMA((3,)),
        ],
        compiler_params=pltpu.CompilerParams(
            collective_id=0, vmem_limit_bytes=_VMEM_LIMIT,
            dimension_semantics=("arbitrary",)),
    )(x2, Wdkv, Wuk, Wuv, Wkr, Wq, Wqr)


def _attention(Q_m, Qr_m, Kr_b, K_b, V_b):
    scale = (Dh + Dr) ** -0.5

    def body(q_ref, qr_ref, kr_ref, k_ref, v_ref, o_ref):
        kr = kr_ref[...]
        dot_t = lambda a, b: lax.dot_general(
            a, b, (((1,), (1,)), ((), ())), preferred_element_type=jnp.float32)
        for i in range(2):
            q = q_ref[:, i * Dh:(i + 1) * Dh]
            k = k_ref[:, i * Dh:(i + 1) * Dh]
            v = v_ref[:, i * Dh:(i + 1) * Dh]
            qr = qr_ref[:, i * Dr:(i + 1) * Dr]
            s = (dot_t(q, k) + dot_t(qr, kr)) * scale
            m = jnp.max(s, axis=-1, keepdims=True)
            p = jnp.exp(s - m)
            p = p / jnp.sum(p, axis=-1, keepdims=True)
            o_ref[:, i * Dh:(i + 1) * Dh] = jnp.dot(
                p, v, preferred_element_type=jnp.float32)

    return pl.pallas_call(
        body,
        grid=(H // 2,),
        in_specs=[
            pl.BlockSpec((MB, 2 * Dh), lambda h: (0, h)),
            pl.BlockSpec((MB, 2 * Dr), lambda h: (0, h)),
            pl.BlockSpec((S, Dr), lambda h: (0, 0)),
            pl.BlockSpec((S, 2 * Dh), lambda h: (0, h)),
            pl.BlockSpec((S, 2 * Dh), lambda h: (0, h)),
        ],
        out_specs=pl.BlockSpec((MB, 2 * Dh), lambda h: (0, h)),
        out_shape=jax.ShapeDtypeStruct((MB, D), jnp.float32),
        compiler_params=_CP,
    )(Q_m, Qr_m, Kr_b, K_b, V_b)


def _matmul(a, w, block_n=512):
    M, K = a.shape
    _, N = w.shape
    block_n = min(block_n, N)

    def body(a_ref, w_ref, o_ref):
        o_ref[...] = jnp.dot(a_ref[...], w_ref[...],
                             preferred_element_type=jnp.float32)

    return pl.pallas_call(
        body,
        grid=(N // block_n,),
        in_specs=[
            pl.BlockSpec((M, K), lambda j: (0, 0)),
            pl.BlockSpec((K, block_n), lambda j: (0, j)),
        ],
        out_specs=pl.BlockSpec((M, block_n), lambda j: (0, j)),
        out_shape=jax.ShapeDtypeStruct((M, N), jnp.float32),
        compiler_params=_CP,
    )(a, w)


def _allgather_out(out_mine):
    SPLITS = ((0, 48), (48, 48), (96, 32))

    def body(in_ref, out_ref, scr, send_sems, recv_sems):
        bx = lax.axis_index("x")
        by = lax.axis_index("y")
        bz = lax.axis_index("z")
        nbrs = [(1 - bx, by, bz), (bx, 1 - by, bz), (bx, by, 1 - bz)]

        def blk(cx, cy, cz):
            return ((cy * 2 + cz) * 2 + cx) * MB

        me = blk(bx, by, bz)
        x_o = blk(1 - bx, by, bz)
        y_o = blk(bx, 1 - by, bz)
        z_o = blk(bx, by, 1 - bz)
        xy_o = blk(1 - bx, 1 - by, bz)
        xz_o = blk(1 - bx, by, 1 - bz)
        yz_o = blk(bx, 1 - by, 1 - bz)

        barrier = pltpu.get_barrier_semaphore()
        for n in nbrs:
            pl.semaphore_signal(barrier, inc=1, device_id=n,
                                device_id_type=_MESH)
        pl.semaphore_wait(barrier, 3)

        scr[pl.ds(me, MB), :] = in_ref[...].astype(jnp.bfloat16)

        def xfer(link, step, src_off, n_rows):
            return pltpu.make_async_remote_copy(
                src_ref=scr.at[pl.ds(src_off, n_rows), :],
                dst_ref=scr.at[pl.ds(src_off, n_rows), :],
                send_sem=send_sems.at[link, step],
                recv_sem=recv_sems.at[link, step],
                device_id=nbrs[link], device_id_type=_MESH)

        s1 = [xfer(l, 0, me, MB) for l in range(3)]
        for r in s1:
            r.start()
        for r in s1:
            r.wait()
        s2 = [xfer(0, 1, y_o, MB), xfer(1, 1, z_o, MB), xfer(2, 1, x_o, MB)]
        for r in s2:
            r.start()
        for r in s2:
            r.wait()
        srcs = (yz_o, xz_o, xy_o)
        s3 = [xfer(l, 2, srcs[l] + SPLITS[l][0], SPLITS[l][1])
              for l in range(3)]
        for r in s3:
            r.start()
        for r in s3:
            r.wait()

        out_ref[...] = scr[...].astype(jnp.float32).reshape(B, S, D)

    return pl.pallas_call(
        body,
        out_shape=jax.ShapeDtypeStruct((B, S, D), jnp.float32),
        in_specs=[pl.BlockSpec(memory_space=pltpu.VMEM)],
        out_specs=pl.BlockSpec(memory_space=pltpu.VMEM),
        scratch_shapes=[
            pltpu.VMEM((B * S, D), jnp.bfloat16),
            pltpu.SemaphoreType.DMA((3, 3)),
            pltpu.SemaphoreType.DMA((3, 3)),
        ],
        compiler_params=pltpu.CompilerParams(
            collective_id=1, vmem_limit_bytes=_VMEM_LIMIT),
    )(out_mine)


def kernel(x, Wdkv, Wuk, Wuv, Wq, Wqr, Wkr, Wo):
    x2 = x.reshape(B * S, D)

    Q_m, Qr_m, Kr_b, K_b, V_b = _proj_exchange(
        x2, Wdkv, Wuk, Wuv, Wkr, Wq, Wqr)

    O_m = _attention(Q_m, Qr_m, Kr_b, K_b, V_b)
    out_mine = _matmul(O_m, Wo)

    return _allgather_out(out_mine)


# device time: 146929 ns/iter; 2.0703x vs baseline; 1.1038x over previous
import jax
import jax.numpy as jnp
from jax import lax
from jax.experimental import pallas as pl
from jax.experimental.pallas import tpu as pltpu

B, S, H, Dh, Dr = 4, 256, 32, 128, 64
D = H * Dh
MB = 128
_VMEM_LIMIT = 100 * 1024 * 1024
_CP = pltpu.CompilerParams(vmem_limit_bytes=_VMEM_LIMIT)
_MESH = pl.DeviceIdType.MESH

NQ_BLK = 16
NQR_BLK = 8
GRID_A = NQ_BLK + NQR_BLK


def _proj_exchange(x2, Wdkv, Wuk, Wuv, Wkr, Wq, Wqr):
    dc = Wdkv.shape[1]

    def body(x_ref, wdkv_ref, wuk_ref, wuv_ref, wkr_ref, wq_ref, wqr_ref,
             q_ref, qr_ref, kr_ref, k_ref, v_ref,
             c_mine, c_peer, wuk_bf, wuv_bf, wuk_p, wuv_p,
             send_sems, recv_sems):
        j = pl.program_id(0)
        my_x = lax.axis_index("x")
        b = lax.axis_index("y") * 2 + lax.axis_index("z")
        peer = (1 - my_x, lax.axis_index("y"), lax.axis_index("z"))
        boff = b * S

        rdma_c = pltpu.make_async_remote_copy(
            src_ref=c_mine, dst_ref=c_peer,
            send_sem=send_sems.at[0], recv_sem=recv_sems.at[0],
            device_id=peer, device_id_type=_MESH)
        rdma_uk = pltpu.make_async_remote_copy(
            src_ref=wuk_bf, dst_ref=wuk_p,
            send_sem=send_sems.at[1], recv_sem=recv_sems.at[1],
            device_id=peer, device_id_type=_MESH)
        rdma_uv = pltpu.make_async_remote_copy(
            src_ref=wuv_bf, dst_ref=wuv_p,
            send_sem=send_sems.at[2], recv_sem=recv_sems.at[2],
            device_id=peer, device_id_type=_MESH)

        @pl.when(j == 0)
        def _start():
            barrier = pltpu.get_barrier_semaphore()
            pl.semaphore_signal(barrier, inc=1, device_id=peer,
                                device_id_type=_MESH)
            pl.semaphore_wait(barrier, 1)
            xb = x_ref[pl.ds(boff, S), :]
            c_mine[...] = jnp.dot(
                xb, wdkv_ref[...],
                preferred_element_type=jnp.float32).astype(jnp.bfloat16)
            wuk_bf[...] = wuk_ref[...].astype(jnp.bfloat16)
            wuv_bf[...] = wuv_ref[...].astype(jnp.bfloat16)
            rdma_c.start()
            rdma_uk.start()
            rdma_uv.start()
            kr_ref[...] = jnp.dot(xb, wkr_ref[...],
                                  preferred_element_type=jnp.float32)

        x_mine = x_ref[pl.ds(boff + my_x * MB, MB), :]

        @pl.when(j < NQ_BLK)
        def _q():
            q_ref[...] = jnp.dot(x_mine, wq_ref[...],
                                 preferred_element_type=jnp.float32)

        @pl.when(j >= NQ_BLK)
        def _qr():
            qr_ref[...] = jnp.dot(x_mine, wqr_ref[...],
                                  preferred_element_type=jnp.float32)

        @pl.when(j == NQ_BLK)
        def _finish():
            rdma_c.wait()
            rdma_uk.wait()
            rdma_uv.wait()
            cm = c_mine[...]
            cp = c_peer[...]
            k_ref[...] = (
                jnp.dot(cm, wuk_bf[...], preferred_element_type=jnp.float32)
                + jnp.dot(cp, wuk_p[...], preferred_element_type=jnp.float32))
            v_ref[...] = (
                jnp.dot(cm, wuv_bf[...], preferred_element_type=jnp.float32)
                + jnp.dot(cp, wuv_p[...], preferred_element_type=jnp.float32))

    full = lambda shape: pl.BlockSpec(shape, lambda j: (0,) * len(shape))
    wq_spec = pl.BlockSpec((D, 256), lambda j: (0, jnp.minimum(j, NQ_BLK - 1)))
    wqr_spec = pl.BlockSpec(
        (D, 256), lambda j: (0, jnp.clip(j - NQ_BLK, 0, NQR_BLK - 1)))

    return pl.pallas_call(
        body,
        grid=(GRID_A,),
        in_specs=[
            full((B * S, D)),
            full((D, dc)),
            full((dc, D)),
            full((dc, D)),
            full((D, Dr)),
            wq_spec,
            wqr_spec,
        ],
        out_specs=(
            pl.BlockSpec((MB, 256), lambda j: (0, jnp.minimum(j, NQ_BLK - 1))),
            pl.BlockSpec((MB, 256),
                         lambda j: (0, jnp.clip(j - NQ_BLK, 0, NQR_BLK - 1))),
            full((S, Dr)),
            full((S, D)),
            full((S, D)),
        ),
        out_shape=(
            jax.ShapeDtypeStruct((MB, D), jnp.float32),
            jax.ShapeDtypeStruct((MB, H * Dr), jnp.float32),
            jax.ShapeDtypeStruct((S, Dr), jnp.float32),
            jax.ShapeDtypeStruct((S, D), jnp.float32),
            jax.ShapeDtypeStruct((S, D), jnp.float32),
        ),
        scratch_shapes=[
            pltpu.VMEM((S, dc), jnp.bfloat16),
            pltpu.VMEM((S, dc), jnp.bfloat16),
            pltpu.VMEM((dc, D), jnp.bfloat16),
            pltpu.VMEM((dc, D), jnp.bfloat16),
            pltpu.VMEM((dc, D), jnp.bfloat16),
            pltpu.VMEM((dc, D), jnp.bfloat16),
            pltpu.SemaphoreType.DMA((3,)),
            pltpu.SemaphoreType.DMA((3,)),
        ],
        compiler_params=pltpu.CompilerParams(
            collective_id=0, vmem_limit_bytes=_VMEM_LIMIT,
            dimension_semantics=("arbitrary",)),
    )(x2, Wdkv, Wuk, Wuv, Wkr, Wq, Wqr)


def _attention(Q_m, Qr_m, Kr_b, K_b, V_b):
    scale = (Dh + Dr) ** -0.5

    def body(q_ref, qr_ref, kr_ref, k_ref, v_ref, o_ref):
        kr = kr_ref[...]
        dot_t = lambda a, b: lax.dot_general(
            a, b, (((1,), (1,)), ((), ())), preferred_element_type=jnp.float32)
        for i in range(2):
            q = q_ref[:, i * Dh:(i + 1) * Dh]
            k = k_ref[:, i * Dh:(i + 1) * Dh]
            v = v_ref[:, i * Dh:(i + 1) * Dh]
            qr = qr_ref[:, i * Dr:(i + 1) * Dr]
            s = (dot_t(q, k) + dot_t(qr, kr)) * scale
            m = jnp.max(s, axis=-1, keepdims=True)
            p = jnp.exp(s - m)
            p = p / jnp.sum(p, axis=-1, keepdims=True)
            o_ref[:, i * Dh:(i + 1) * Dh] = jnp.dot(
                p, v, preferred_element_type=jnp.float32)

    return pl.pallas_call(
        body,
        grid=(H // 2,),
        in_specs=[
            pl.BlockSpec((MB, 2 * Dh), lambda h: (0, h)),
            pl.BlockSpec((MB, 2 * Dr), lambda h: (0, h)),
            pl.BlockSpec((S, Dr), lambda h: (0, 0)),
            pl.BlockSpec((S, 2 * Dh), lambda h: (0, h)),
            pl.BlockSpec((S, 2 * Dh), lambda h: (0, h)),
        ],
        out_specs=pl.BlockSpec((MB, 2 * Dh), lambda h: (0, h)),
        out_shape=jax.ShapeDtypeStruct((MB, D), jnp.float32),
        compiler_params=_CP,
    )(Q_m, Qr_m, Kr_b, K_b, V_b)


def _matmul(a, w, block_n=512):
    M, K = a.shape
    _, N = w.shape
    block_n = min(block_n, N)

    def body(a_ref, w_ref, o_ref):
        o_ref[...] = jnp.dot(a_ref[...], w_ref[...],
                             preferred_element_type=jnp.float32)

    return pl.pallas_call(
        body,
        grid=(N // block_n,),
        in_specs=[
            pl.BlockSpec((M, K), lambda j: (0, 0)),
            pl.BlockSpec((K, block_n), lambda j: (0, j)),
        ],
        out_specs=pl.BlockSpec((M, block_n), lambda j: (0, j)),
        out_shape=jax.ShapeDtypeStruct((M, N), jnp.float32),
        compiler_params=_CP,
    )(a, w)


NO_BLK = 8


def _out_allgather(O_m, Wo):
    BN = D // NO_BLK
    SPLITS = ((0, 48), (48, 48), (96, 32))

    def body(o_ref, wo_ref, out_ref, scr, send_sems, recv_sems):
        j = pl.program_id(0)
        bx = lax.axis_index("x")
        by = lax.axis_index("y")
        bz = lax.axis_index("z")
        nbrs = [(1 - bx, by, bz), (bx, 1 - by, bz), (bx, by, 1 - bz)]

        def blk(cx, cy, cz):
            return ((cy * 2 + cz) * 2 + cx) * MB

        me = blk(bx, by, bz)
        x_o = blk(1 - bx, by, bz)
        y_o = blk(bx, 1 - by, bz)
        z_o = blk(bx, by, 1 - bz)
        xy_o = blk(1 - bx, 1 - by, bz)
        xz_o = blk(1 - bx, by, 1 - bz)
        yz_o = blk(bx, 1 - by, 1 - bz)
        anti_o = blk(1 - bx, 1 - by, 1 - bz)

        def store_f32(row_off, n_rows, col_off, n_cols, val=None):
            bi = row_off // S
            ri = row_off % S
            if val is None:
                val = scr[pl.ds(row_off, n_rows),
                          pl.ds(col_off, n_cols)].astype(jnp.float32)
            out_ref[bi, pl.ds(ri, n_rows), pl.ds(col_off, n_cols)] = val

        def xfer(link, slot, src_off, n_rows, col_off=0, n_cols=D):
            return pltpu.make_async_remote_copy(
                src_ref=scr.at[pl.ds(src_off, n_rows), pl.ds(col_off, n_cols)],
                dst_ref=scr.at[pl.ds(src_off, n_rows), pl.ds(col_off, n_cols)],
                send_sem=send_sems.at[link, slot],
                recv_sem=recv_sems.at[link, slot],
                device_id=nbrs[link], device_id_type=_MESH)

        @pl.when(j == 0)
        def _barrier():
            barrier = pltpu.get_barrier_semaphore()
            for n in nbrs:
                pl.semaphore_signal(barrier, inc=1, device_id=n,
                                    device_id_type=_MESH)
            pl.semaphore_wait(barrier, 3)

        chunk = jnp.dot(o_ref[...], wo_ref[...],
                        preferred_element_type=jnp.float32)
        store_f32(me, MB, j * BN, BN, val=chunk)
        scr[pl.ds(me, MB), pl.ds(j * BN, BN)] = chunk.astype(jnp.bfloat16)
        for l in range(3):
            xfer(l, j, me, MB, j * BN, BN).start()

        @pl.when(j == NO_BLK - 1)
        def _tail():
            for l in range(3):
                for jj in range(NO_BLK):
                    xfer(l, jj, me, MB, jj * BN, BN).wait()
            s2 = [xfer(0, NO_BLK, y_o, MB), xfer(1, NO_BLK, z_o, MB),
                  xfer(2, NO_BLK, x_o, MB)]
            for r in s2:
                r.start()
            for off in (x_o, y_o, z_o):
                store_f32(off, MB, 0, D)
            for r in s2:
                r.wait()
            srcs = (yz_o, xz_o, xy_o)
            s3 = [xfer(l, NO_BLK + 1, srcs[l] + SPLITS[l][0], SPLITS[l][1])
                  for l in range(3)]
            for r in s3:
                r.start()
            for off in (xy_o, xz_o, yz_o):
                store_f32(off, MB, 0, D)
            for r in s3:
                r.wait()
            store_f32(anti_o, MB, 0, D)

    return pl.pallas_call(
        body,
        grid=(NO_BLK,),
        in_specs=[
            pl.BlockSpec((MB, D), lambda j: (0, 0)),
            pl.BlockSpec((D, BN), lambda j: (0, j)),
        ],
        out_specs=pl.BlockSpec((B, S, D), lambda j: (0, 0, 0)),
        out_shape=jax.ShapeDtypeStruct((B, S, D), jnp.float32),
        scratch_shapes=[
            pltpu.VMEM((B * S, D), jnp.bfloat16),
            pltpu.SemaphoreType.DMA((3, NO_BLK + 2)),
            pltpu.SemaphoreType.DMA((3, NO_BLK + 2)),
        ],
        compiler_params=pltpu.CompilerParams(
            collective_id=1, vmem_limit_bytes=_VMEM_LIMIT,
            dimension_semantics=("arbitrary",)),
    )(O_m, Wo)


def kernel(x, Wdkv, Wuk, Wuv, Wq, Wqr, Wkr, Wo):
    x2 = x.reshape(B * S, D)

    Q_m, Qr_m, Kr_b, K_b, V_b = _proj_exchange(
        x2, Wdkv, Wuk, Wuv, Wkr, Wq, Wqr)

    O_m = _attention(Q_m, Qr_m, Kr_b, K_b, V_b)
    return _out_allgather(O_m, Wo)


# device time: 136859 ns/iter; 2.2226x vs baseline; 1.0736x over previous
import jax
import jax.numpy as jnp
from jax import lax
from jax.experimental import pallas as pl
from jax.experimental.pallas import tpu as pltpu

B, S, H, Dh, Dr = 4, 256, 32, 128, 64
D = H * Dh
MB = 128
_VMEM_LIMIT = 100 * 1024 * 1024
_CP = pltpu.CompilerParams(vmem_limit_bytes=_VMEM_LIMIT)
_MESH = pl.DeviceIdType.MESH

NQ_BLK = 16
NQR_BLK = 8
GRID_A = NQ_BLK + NQR_BLK


def _proj_exchange(x2, Wdkv, Wuk, Wuv, Wkr, Wq, Wqr):
    dc = Wdkv.shape[1]

    def body(x_ref, wdkv_ref, wuk_ref, wuv_ref, wkr_ref, wq_ref, wqr_ref,
             q_ref, qr_ref, kr_ref, k_ref, v_ref,
             c_mine, c_peer, wuk_bf, wuv_bf, wuk_p, wuv_p,
             send_sems, recv_sems):
        j = pl.program_id(0)
        my_x = lax.axis_index("x")
        b = lax.axis_index("y") * 2 + lax.axis_index("z")
        peer = (1 - my_x, lax.axis_index("y"), lax.axis_index("z"))
        boff = b * S

        rdma_c = pltpu.make_async_remote_copy(
            src_ref=c_mine, dst_ref=c_peer,
            send_sem=send_sems.at[0], recv_sem=recv_sems.at[0],
            device_id=peer, device_id_type=_MESH)
        rdma_uk = pltpu.make_async_remote_copy(
            src_ref=wuk_bf, dst_ref=wuk_p,
            send_sem=send_sems.at[1], recv_sem=recv_sems.at[1],
            device_id=peer, device_id_type=_MESH)
        rdma_uv = pltpu.make_async_remote_copy(
            src_ref=wuv_bf, dst_ref=wuv_p,
            send_sem=send_sems.at[2], recv_sem=recv_sems.at[2],
            device_id=peer, device_id_type=_MESH)

        @pl.when(j == 0)
        def _start():
            barrier = pltpu.get_barrier_semaphore()
            pl.semaphore_signal(barrier, inc=1, device_id=peer,
                                device_id_type=_MESH)
            pl.semaphore_wait(barrier, 1)
            xb = x_ref[pl.ds(boff, S), :]
            c_mine[...] = jnp.dot(
                xb, wdkv_ref[...],
                preferred_element_type=jnp.float32).astype(jnp.bfloat16)
            wuk_bf[...] = wuk_ref[...].astype(jnp.bfloat16)
            wuv_bf[...] = wuv_ref[...].astype(jnp.bfloat16)
            rdma_c.start()
            rdma_uk.start()
            rdma_uv.start()
            kr_ref[...] = jnp.dot(xb, wkr_ref[...],
                                  preferred_element_type=jnp.float32)

        x_mine = x_ref[pl.ds(boff + my_x * MB, MB), :]

        @pl.when(j < NQ_BLK)
        def _q():
            q_ref[...] = jnp.dot(x_mine, wq_ref[...],
                                 preferred_element_type=jnp.float32)

        @pl.when(j >= NQ_BLK)
        def _qr():
            qr_ref[...] = jnp.dot(x_mine, wqr_ref[...],
                                  preferred_element_type=jnp.float32)

        @pl.when(j == NQ_BLK)
        def _finish():
            rdma_c.wait()
            rdma_uk.wait()
            rdma_uv.wait()
            cm = c_mine[...]
            cp = c_peer[...]
            k_ref[...] = (
                jnp.dot(cm, wuk_bf[...], preferred_element_type=jnp.float32)
                + jnp.dot(cp, wuk_p[...], preferred_element_type=jnp.float32))
            v_ref[...] = (
                jnp.dot(cm, wuv_bf[...], preferred_element_type=jnp.float32)
                + jnp.dot(cp, wuv_p[...], preferred_element_type=jnp.float32))

    full = lambda shape: pl.BlockSpec(shape, lambda j: (0,) * len(shape))
    wq_spec = pl.BlockSpec((D, 256), lambda j: (0, jnp.minimum(j, NQ_BLK - 1)))
    wqr_spec = pl.BlockSpec(
        (D, 256), lambda j: (0, jnp.clip(j - NQ_BLK, 0, NQR_BLK - 1)))

    return pl.pallas_call(
        body,
        grid=(GRID_A,),
        in_specs=[
            full((B * S, D)),
            full((D, dc)),
            full((dc, D)),
            full((dc, D)),
            full((D, Dr)),
            wq_spec,
            wqr_spec,
        ],
        out_specs=(
            pl.BlockSpec((MB, 256), lambda j: (0, jnp.minimum(j, NQ_BLK - 1))),
            pl.BlockSpec((MB, 256),
                         lambda j: (0, jnp.clip(j - NQ_BLK, 0, NQR_BLK - 1))),
            full((S, Dr)),
            full((S, D)),
            full((S, D)),
        ),
        out_shape=(
            jax.ShapeDtypeStruct((MB, D), jnp.float32),
            jax.ShapeDtypeStruct((MB, H * Dr), jnp.float32),
            jax.ShapeDtypeStruct((S, Dr), jnp.float32),
            jax.ShapeDtypeStruct((S, D), jnp.float32),
            jax.ShapeDtypeStruct((S, D), jnp.float32),
        ),
        scratch_shapes=[
            pltpu.VMEM((S, dc), jnp.bfloat16),
            pltpu.VMEM((S, dc), jnp.bfloat16),
            pltpu.VMEM((dc, D), jnp.bfloat16),
            pltpu.VMEM((dc, D), jnp.bfloat16),
            pltpu.VMEM((dc, D), jnp.bfloat16),
            pltpu.VMEM((dc, D), jnp.bfloat16),
            pltpu.SemaphoreType.DMA((3,)),
            pltpu.SemaphoreType.DMA((3,)),
        ],
        compiler_params=pltpu.CompilerParams(
            collective_id=0, vmem_limit_bytes=_VMEM_LIMIT,
            dimension_semantics=("arbitrary",)),
    )(x2, Wdkv, Wuk, Wuv, Wkr, Wq, Wqr)


def _attention(Q_m, Qr_m, Kr_b, K_b, V_b):
    scale = (Dh + Dr) ** -0.5

    def body(q_ref, qr_ref, kr_ref, k_ref, v_ref, o_ref):
        kr = kr_ref[...]
        dot_t = lambda a, b: lax.dot_general(
            a, b, (((1,), (1,)), ((), ())), preferred_element_type=jnp.float32)
        for i in range(4):
            q = q_ref[:, i * Dh:(i + 1) * Dh]
            k = k_ref[:, i * Dh:(i + 1) * Dh]
            v = v_ref[:, i * Dh:(i + 1) * Dh]
            qr = qr_ref[:, i * Dr:(i + 1) * Dr]
            s = (dot_t(q, k) + dot_t(qr, kr)) * scale
            p = jnp.exp(s)
            p = p / jnp.sum(p, axis=-1, keepdims=True)
            o_ref[:, i * Dh:(i + 1) * Dh] = jnp.dot(
                p, v, preferred_element_type=jnp.float32)

    return pl.pallas_call(
        body,
        grid=(H // 4,),
        in_specs=[
            pl.BlockSpec((MB, 4 * Dh), lambda h: (0, h)),
            pl.BlockSpec((MB, 4 * Dr), lambda h: (0, h)),
            pl.BlockSpec((S, Dr), lambda h: (0, 0)),
            pl.BlockSpec((S, 4 * Dh), lambda h: (0, h)),
            pl.BlockSpec((S, 4 * Dh), lambda h: (0, h)),
        ],
        out_specs=pl.BlockSpec((MB, 4 * Dh), lambda h: (0, h)),
        out_shape=jax.ShapeDtypeStruct((MB, D), jnp.float32),
        compiler_params=_CP,
    )(Q_m, Qr_m, Kr_b, K_b, V_b)


def _matmul(a, w, block_n=512):
    M, K = a.shape
    _, N = w.shape
    block_n = min(block_n, N)

    def body(a_ref, w_ref, o_ref):
        o_ref[...] = jnp.dot(a_ref[...], w_ref[...],
                             preferred_element_type=jnp.float32)

    return pl.pallas_call(
        body,
        grid=(N // block_n,),
        in_specs=[
            pl.BlockSpec((M, K), lambda j: (0, 0)),
            pl.BlockSpec((K, block_n), lambda j: (0, j)),
        ],
        out_specs=pl.BlockSpec((M, block_n), lambda j: (0, j)),
        out_shape=jax.ShapeDtypeStruct((M, N), jnp.float32),
        compiler_params=_CP,
    )(a, w)


NO_BLK = 8


def _out_allgather(O_m, Wo):
    BN = D // NO_BLK
    SPLITS = ((0, 48), (48, 48), (96, 32))

    def body(o_ref, wo_ref, out_ref, scr, stage_c, stage_b,
             csem, bsem, send_sems, recv_sems):
        j = pl.program_id(0)
        bx = lax.axis_index("x")
        by = lax.axis_index("y")
        bz = lax.axis_index("z")
        nbrs = [(1 - bx, by, bz), (bx, 1 - by, bz), (bx, by, 1 - bz)]

        def blk(cx, cy, cz):
            return ((cy * 2 + cz) * 2 + cx) * MB

        me = blk(bx, by, bz)
        x_o = blk(1 - bx, by, bz)
        y_o = blk(bx, 1 - by, bz)
        z_o = blk(bx, by, 1 - bz)
        xy_o = blk(1 - bx, 1 - by, bz)
        xz_o = blk(1 - bx, by, 1 - bz)
        yz_o = blk(bx, 1 - by, 1 - bz)
        anti_o = blk(1 - bx, 1 - by, 1 - bz)

        def chunk_copy(slot, col_off):
            return pltpu.make_async_copy(
                stage_c.at[slot],
                out_ref.at[me // S, pl.ds(me % S, MB), pl.ds(col_off, BN)],
                csem.at[slot])

        def block_copy(slot, row_off):
            return pltpu.make_async_copy(
                stage_b.at[slot],
                out_ref.at[row_off // S, pl.ds(row_off % S, MB), :],
                bsem.at[slot])

        def xfer(link, slot, src_off, n_rows, col_off=0, n_cols=D):
            return pltpu.make_async_remote_copy(
                src_ref=scr.at[pl.ds(src_off, n_rows), pl.ds(col_off, n_cols)],
                dst_ref=scr.at[pl.ds(src_off, n_rows), pl.ds(col_off, n_cols)],
                send_sem=send_sems.at[link, slot],
                recv_sem=recv_sems.at[link, slot],
                device_id=nbrs[link], device_id_type=_MESH)

        @pl.when(j == 0)
        def _barrier():
            barrier = pltpu.get_barrier_semaphore()
            for n in nbrs:
                pl.semaphore_signal(barrier, inc=1, device_id=n,
                                    device_id_type=_MESH)
            pl.semaphore_wait(barrier, 3)

        chunk = jnp.dot(o_ref[...], wo_ref[...],
                        preferred_element_type=jnp.float32)
        slot = lax.rem(j, 2)

        @pl.when(j >= 2)
        def _reuse():
            chunk_copy(slot, (j - 2) * BN).wait()

        stage_c[slot] = chunk
        chunk_copy(slot, j * BN).start()
        scr[pl.ds(me, MB), pl.ds(j * BN, BN)] = chunk.astype(jnp.bfloat16)
        for l in range(3):
            xfer(l, j, me, MB, j * BN, BN).start()

        @pl.when(j == NO_BLK - 1)
        def _tail():
            for l in range(3):
                for jj in range(NO_BLK):
                    xfer(l, jj, me, MB, jj * BN, BN).wait()
            s2 = [xfer(0, NO_BLK, y_o, MB), xfer(1, NO_BLK, z_o, MB),
                  xfer(2, NO_BLK, x_o, MB)]
            for r in s2:
                r.start()

            def stage_out(offs, base):
                for i, off in enumerate(offs):
                    sl = (base + i) % 2
                    stage_b[sl] = scr[pl.ds(off, MB), :].astype(jnp.float32)
                    block_copy(sl, off).start()

            def wait_out(offs, base):
                for i, off in enumerate(offs):
                    block_copy((base + i) % 2, off).wait()

            stage_out((x_o, y_o), 0)
            wait_out((x_o, y_o), 0)
            stage_out((z_o,), 0)
            for r in s2:
                r.wait()
            srcs = (yz_o, xz_o, xy_o)
            s3 = [xfer(l, NO_BLK + 1, srcs[l] + SPLITS[l][0], SPLITS[l][1])
                  for l in range(3)]
            for r in s3:
                r.start()
            wait_out((z_o,), 0)
            stage_out((xy_o, xz_o), 1)
            wait_out((xy_o, xz_o), 1)
            stage_out((yz_o,), 1)
            for r in s3:
                r.wait()
            wait_out((yz_o,), 1)
            stage_out((anti_o,), 0)
            wait_out((anti_o,), 0)
            chunk_copy(0, (NO_BLK - 2) * BN).wait()
            chunk_copy(1, (NO_BLK - 1) * BN).wait()

    return pl.pallas_call(
        body,
        grid=(NO_BLK,),
        in_specs=[
            pl.BlockSpec((MB, D), lambda j: (0, 0)),
            pl.BlockSpec((D, BN), lambda j: (0, j)),
        ],
        out_specs=pl.BlockSpec(memory_space=pltpu.MemorySpace.HBM),
        out_shape=jax.ShapeDtypeStruct((B, S, D), jnp.float32),
        scratch_shapes=[
            pltpu.VMEM((B * S, D), jnp.bfloat16),
            pltpu.VMEM((2, MB, D // NO_BLK), jnp.float32),
            pltpu.VMEM((2, MB, D), jnp.float32),
            pltpu.SemaphoreType.DMA((2,)),
            pltpu.SemaphoreType.DMA((2,)),
            pltpu.SemaphoreType.DMA((3, NO_BLK + 2)),
            pltpu.SemaphoreType.DMA((3, NO_BLK + 2)),
        ],
        compiler_params=pltpu.CompilerParams(
            collective_id=1, vmem_limit_bytes=_VMEM_LIMIT,
            dimension_semantics=("arbitrary",)),
    )(O_m, Wo)


def kernel(x, Wdkv, Wuk, Wuv, Wq, Wqr, Wkr, Wo):
    x2 = x.reshape(B * S, D)

    Q_m, Qr_m, Kr_b, K_b, V_b = _proj_exchange(
        x2, Wdkv, Wuk, Wuv, Wkr, Wq, Wqr)

    O_m = _attention(Q_m, Qr_m, Kr_b, K_b, V_b)
    return _out_allgather(O_m, Wo)


# device time: 132605 ns/iter; 2.2939x vs baseline; 1.0321x over previous
import jax
import jax.numpy as jnp
from jax import lax
from jax.experimental import pallas as pl
from jax.experimental.pallas import tpu as pltpu

B, S, H, Dh, Dr = 4, 256, 32, 128, 64
D = H * Dh
MB = 128
_VMEM_LIMIT = 100 * 1024 * 1024
_CP = pltpu.CompilerParams(vmem_limit_bytes=_VMEM_LIMIT)
_MESH = pl.DeviceIdType.MESH

NQ_BLK = 16
NQR_BLK = 8
GRID_A = NQ_BLK + NQR_BLK


def _proj_attn(x2, Wdkv, Wuk, Wuv, Wkr, Wq, Wqr):
    dc = Wdkv.shape[1]
    scale = (Dh + Dr) ** -0.5

    def body(x_ref, wdkv_ref, wuk_ref, wuv_ref, wkr_ref, wq_ref, wqr_ref,
             o_ref,
             q_scr, kr_scr, k_scr, v_scr,
             c_mine, c_peer, wuk_bf, wuv_bf, wuk_p, wuv_p,
             send_sems, recv_sems):
        j = pl.program_id(0)
        my_x = lax.axis_index("x")
        b = lax.axis_index("y") * 2 + lax.axis_index("z")
        peer = (1 - my_x, lax.axis_index("y"), lax.axis_index("z"))
        boff = b * S

        rdma_c = pltpu.make_async_remote_copy(
            src_ref=c_mine, dst_ref=c_peer,
            send_sem=send_sems.at[0], recv_sem=recv_sems.at[0],
            device_id=peer, device_id_type=_MESH)
        rdma_uk = pltpu.make_async_remote_copy(
            src_ref=wuk_bf, dst_ref=wuk_p,
            send_sem=send_sems.at[1], recv_sem=recv_sems.at[1],
            device_id=peer, device_id_type=_MESH)
        rdma_uv = pltpu.make_async_remote_copy(
            src_ref=wuv_bf, dst_ref=wuv_p,
            send_sem=send_sems.at[2], recv_sem=recv_sems.at[2],
            device_id=peer, device_id_type=_MESH)

        @pl.when(j == 0)
        def _start():
            barrier = pltpu.get_barrier_semaphore()
            pl.semaphore_signal(barrier, inc=1, device_id=peer,
                                device_id_type=_MESH)
            pl.semaphore_wait(barrier, 1)
            xb = x_ref[pl.ds(boff, S), :]
            c_mine[...] = jnp.dot(
                xb, wdkv_ref[...],
                preferred_element_type=jnp.float32).astype(jnp.bfloat16)
            wuk_bf[...] = wuk_ref[...].astype(jnp.bfloat16)
            wuv_bf[...] = wuv_ref[...].astype(jnp.bfloat16)
            rdma_c.start()
            rdma_uk.start()
            rdma_uv.start()
            kr_scr[...] = jnp.dot(xb, wkr_ref[...],
                                  preferred_element_type=jnp.float32)

        x_mine = x_ref[pl.ds(boff + my_x * MB, MB), :]

        @pl.when(j < NQ_BLK)
        def _q():
            chunk = jnp.dot(x_mine, wq_ref[...],
                            preferred_element_type=jnp.float32)
            q_scr[2 * j] = chunk[:, :Dh]
            q_scr[2 * j + 1] = chunk[:, Dh:]

        @pl.when(j == 12)
        def _combine():
            rdma_c.wait()
            rdma_uk.wait()
            rdma_uv.wait()
            cm = c_mine[...]
            cp = c_peer[...]
            for g in range(H // 4):
                cs = slice(g * 4 * Dh, (g + 1) * 4 * Dh)
                k4 = (jnp.dot(cm, wuk_bf[:, cs],
                              preferred_element_type=jnp.float32)
                      + jnp.dot(cp, wuk_p[:, cs],
                                preferred_element_type=jnp.float32))
                v4 = (jnp.dot(cm, wuv_bf[:, cs],
                              preferred_element_type=jnp.float32)
                      + jnp.dot(cp, wuv_p[:, cs],
                                preferred_element_type=jnp.float32))
                for i in range(4):
                    k_scr[4 * g + i] = k4[:, i * Dh:(i + 1) * Dh]
                    v_scr[4 * g + i] = v4[:, i * Dh:(i + 1) * Dh]

        @pl.when(j >= NQ_BLK)
        def _attn():
            qr_chunk = jnp.dot(x_mine, wqr_ref[...],
                               preferred_element_type=jnp.float32)
            g = (j - NQ_BLK) * 4
            kr = kr_scr[...]
            dot_t = lambda a, b: lax.dot_general(
                a, b, (((1,), (1,)), ((), ())),
                preferred_element_type=jnp.float32)
            for i in range(4):
                q = q_scr[g + i]
                k = k_scr[g + i]
                v = v_scr[g + i]
                qr = qr_chunk[:, i * Dr:(i + 1) * Dr]
                s = (dot_t(q, k) + dot_t(qr, kr)) * scale
                p = jnp.exp(s)
                p = p / jnp.sum(p, axis=-1, keepdims=True)
                o_ref[:, i * Dh:(i + 1) * Dh] = jnp.dot(
                    p, v, preferred_element_type=jnp.float32)

    full = lambda shape: pl.BlockSpec(shape, lambda j: (0,) * len(shape))
    wq_spec = pl.BlockSpec((D, 256), lambda j: (0, jnp.minimum(j, NQ_BLK - 1)))
    wqr_spec = pl.BlockSpec(
        (D, 256), lambda j: (0, jnp.clip(j - NQ_BLK, 0, NQR_BLK - 1)))

    return pl.pallas_call(
        body,
        grid=(GRID_A,),
        in_specs=[
            full((B * S, D)),
            full((D, dc)),
            full((dc, D)),
            full((dc, D)),
            full((D, Dr)),
            wq_spec,
            wqr_spec,
        ],
        out_specs=pl.BlockSpec(
            (MB, 4 * Dh), lambda j: (0, jnp.clip(j - NQ_BLK, 0, NQR_BLK - 1))),
        out_shape=jax.ShapeDtypeStruct((MB, D), jnp.float32),
        scratch_shapes=[
            pltpu.VMEM((H, MB, Dh), jnp.float32),
            pltpu.VMEM((S, Dr), jnp.float32),
            pltpu.VMEM((H, S, Dh), jnp.float32),
            pltpu.VMEM((H, S, Dh), jnp.float32),
            pltpu.VMEM((S, dc), jnp.bfloat16),
            pltpu.VMEM((S, dc), jnp.bfloat16),
            pltpu.VMEM((dc, D), jnp.bfloat16),
            pltpu.VMEM((dc, D), jnp.bfloat16),
            pltpu.VMEM((dc, D), jnp.bfloat16),
            pltpu.VMEM((dc, D), jnp.bfloat16),
            pltpu.SemaphoreType.DMA((3,)),
            pltpu.SemaphoreType.DMA((3,)),
        ],
        compiler_params=pltpu.CompilerParams(
            collective_id=0, vmem_limit_bytes=_VMEM_LIMIT,
            dimension_semantics=("arbitrary",)),
    )(x2, Wdkv, Wuk, Wuv, Wkr, Wq, Wqr)


NO_BLK = 8


def _out_allgather(O_m, Wo):
    BN = D // NO_BLK
    SPLITS = ((0, 48), (48, 48), (96, 32))

    def body(o_ref, wo_ref, out_ref, scr, stage_c, stage_b,
             csem, bsem, send_sems, recv_sems):
        j = pl.program_id(0)
        bx = lax.axis_index("x")
        by = lax.axis_index("y")
        bz = lax.axis_index("z")
        nbrs = [(1 - bx, by, bz), (bx, 1 - by, bz), (bx, by, 1 - bz)]

        def blk(cx, cy, cz):
            return ((cy * 2 + cz) * 2 + cx) * MB

        me = blk(bx, by, bz)
        x_o = blk(1 - bx, by, bz)
        y_o = blk(bx, 1 - by, bz)
        z_o = blk(bx, by, 1 - bz)
        xy_o = blk(1 - bx, 1 - by, bz)
        xz_o = blk(1 - bx, by, 1 - bz)
        yz_o = blk(bx, 1 - by, 1 - bz)
        anti_o = blk(1 - bx, 1 - by, 1 - bz)

        def chunk_copy(slot, col_off):
            return pltpu.make_async_copy(
                stage_c.at[slot],
                out_ref.at[me // S, pl.ds(me % S, MB), pl.ds(col_off, BN)],
                csem.at[slot])

        def block_copy(slot, row_off):
            return pltpu.make_async_copy(
                stage_b.at[slot],
                out_ref.at[row_off // S, pl.ds(row_off % S, MB), :],
                bsem.at[slot])

        def xfer(link, slot, src_off, n_rows, col_off=0, n_cols=D):
            return pltpu.make_async_remote_copy(
                src_ref=scr.at[pl.ds(src_off, n_rows), pl.ds(col_off, n_cols)],
                dst_ref=scr.at[pl.ds(src_off, n_rows), pl.ds(col_off, n_cols)],
                send_sem=send_sems.at[link, slot],
                recv_sem=recv_sems.at[link, slot],
                device_id=nbrs[link], device_id_type=_MESH)

        @pl.when(j == 0)
        def _barrier():
            barrier = pltpu.get_barrier_semaphore()
            for n in nbrs:
                pl.semaphore_signal(barrier, inc=1, device_id=n,
                                    device_id_type=_MESH)
            pl.semaphore_wait(barrier, 3)

        chunk = jnp.dot(o_ref[...], wo_ref[...],
                        preferred_element_type=jnp.float32)
        slot = lax.rem(j, 2)

        @pl.when(j >= 2)
        def _reuse():
            chunk_copy(slot, (j - 2) * BN).wait()

        stage_c[slot] = chunk
        chunk_copy(slot, j * BN).start()
        scr[pl.ds(me, MB), pl.ds(j * BN, BN)] = chunk.astype(jnp.bfloat16)
        for l in range(3):
            xfer(l, j, me, MB, j * BN, BN).start()

        @pl.when(j == NO_BLK - 1)
        def _tail():
            for l in range(3):
                for jj in range(NO_BLK):
                    xfer(l, jj, me, MB, jj * BN, BN).wait()
            s2 = [xfer(0, NO_BLK, y_o, MB), xfer(1, NO_BLK, z_o, MB),
                  xfer(2, NO_BLK, x_o, MB)]
            for r in s2:
                r.start()

            def stage_out(offs, base):
                for i, off in enumerate(offs):
                    sl = (base + i) % 2
                    stage_b[sl] = scr[pl.ds(off, MB), :].astype(jnp.float32)
                    block_copy(sl, off).start()

            def wait_out(offs, base):
                for i, off in enumerate(offs):
                    block_copy((base + i) % 2, off).wait()

            stage_out((x_o, y_o), 0)
            wait_out((x_o, y_o), 0)
            stage_out((z_o,), 0)
            for r in s2:
                r.wait()
            srcs = (yz_o, xz_o, xy_o)
            s3 = [xfer(l, NO_BLK + 1, srcs[l] + SPLITS[l][0], SPLITS[l][1])
                  for l in range(3)]
            for r in s3:
                r.start()
            wait_out((z_o,), 0)
            stage_out((xy_o, xz_o), 1)
            wait_out((xy_o, xz_o), 1)
            stage_out((yz_o,), 1)
            for r in s3:
                r.wait()
            wait_out((yz_o,), 1)
            stage_out((anti_o,), 0)
            wait_out((anti_o,), 0)
            chunk_copy(0, (NO_BLK - 2) * BN).wait()
            chunk_copy(1, (NO_BLK - 1) * BN).wait()

    return pl.pallas_call(
        body,
        grid=(NO_BLK,),
        in_specs=[
            pl.BlockSpec((MB, D), lambda j: (0, 0)),
            pl.BlockSpec((D, BN), lambda j: (0, j)),
        ],
        out_specs=pl.BlockSpec(memory_space=pltpu.MemorySpace.HBM),
        out_shape=jax.ShapeDtypeStruct((B, S, D), jnp.float32),
        scratch_shapes=[
            pltpu.VMEM((B * S, D), jnp.bfloat16),
            pltpu.VMEM((2, MB, D // NO_BLK), jnp.float32),
            pltpu.VMEM((2, MB, D), jnp.float32),
            pltpu.SemaphoreType.DMA((2,)),
            pltpu.SemaphoreType.DMA((2,)),
            pltpu.SemaphoreType.DMA((3, NO_BLK + 2)),
            pltpu.SemaphoreType.DMA((3, NO_BLK + 2)),
        ],
        compiler_params=pltpu.CompilerParams(
            collective_id=1, vmem_limit_bytes=_VMEM_LIMIT,
            dimension_semantics=("arbitrary",)),
    )(O_m, Wo)


def kernel(x, Wdkv, Wuk, Wuv, Wq, Wqr, Wkr, Wo):
    x2 = x.reshape(B * S, D)
    O_m = _proj_attn(x2, Wdkv, Wuk, Wuv, Wkr, Wq, Wqr)
    return _out_allgather(O_m, Wo)


# device time: 122941 ns/iter; 2.4742x vs baseline; 1.0786x over previous
import jax
import jax.numpy as jnp
from jax import lax
from jax.experimental import pallas as pl
from jax.experimental.pallas import tpu as pltpu

B, S, H, Dh, Dr = 4, 256, 32, 128, 64
D = H * Dh
MB = 128
_VMEM_LIMIT = 100 * 1024 * 1024
_CP = pltpu.CompilerParams(vmem_limit_bytes=_VMEM_LIMIT)
_MESH = pl.DeviceIdType.MESH

NQ_BLK = 16
NQR_BLK = 8
GRID_A = NQ_BLK + NQR_BLK


def _proj_attn(x2, Wdkv, Wuk, Wuv, Wkr, Wq, Wqr):
    dc = Wdkv.shape[1]
    scale = (Dh + Dr) ** -0.5

    def body(x_ref, wdkv_ref, wuk_ref, wuv_ref, wkr_ref, wq_ref, wqr_ref,
             o_ref,
             q_scr, kr_scr, k_scr, v_scr,
             c_mine, c_peer, wuk_bf, wuv_bf, wuk_p, wuv_p,
             send_sems, recv_sems):
        j = pl.program_id(0)
        my_x = lax.axis_index("x")
        b = lax.axis_index("y") * 2 + lax.axis_index("z")
        peer = (1 - my_x, lax.axis_index("y"), lax.axis_index("z"))
        boff = b * S

        rdma_c = pltpu.make_async_remote_copy(
            src_ref=c_mine, dst_ref=c_peer,
            send_sem=send_sems.at[0], recv_sem=recv_sems.at[0],
            device_id=peer, device_id_type=_MESH)
        rdma_uk = pltpu.make_async_remote_copy(
            src_ref=wuk_bf, dst_ref=wuk_p,
            send_sem=send_sems.at[1], recv_sem=recv_sems.at[1],
            device_id=peer, device_id_type=_MESH)
        rdma_uv = pltpu.make_async_remote_copy(
            src_ref=wuv_bf, dst_ref=wuv_p,
            send_sem=send_sems.at[2], recv_sem=recv_sems.at[2],
            device_id=peer, device_id_type=_MESH)

        @pl.when(j == 0)
        def _start():
            barrier = pltpu.get_barrier_semaphore()
            pl.semaphore_signal(barrier, inc=1, device_id=peer,
                                device_id_type=_MESH)
            pl.semaphore_wait(barrier, 1)
            xb = x_ref[pl.ds(boff, S), :]
            c_mine[...] = jnp.dot(
                xb, wdkv_ref[...],
                preferred_element_type=jnp.float32).astype(jnp.bfloat16)
            wuk_bf[...] = wuk_ref[...].astype(jnp.bfloat16)
            wuv_bf[...] = wuv_ref[...].astype(jnp.bfloat16)
            rdma_c.start()
            rdma_uk.start()
            rdma_uv.start()
            kr_scr[...] = jnp.dot(xb, wkr_ref[...],
                                  preferred_element_type=jnp.float32)

        x_mine = x_ref[pl.ds(boff + my_x * MB, MB), :]

        @pl.when(j < NQ_BLK)
        def _q():
            chunk = jnp.dot(x_mine, wq_ref[...],
                            preferred_element_type=jnp.float32)
            q_scr[2 * j] = chunk[:, :Dh]
            q_scr[2 * j + 1] = chunk[:, Dh:]

        @pl.when(j == 12)
        def _combine():
            rdma_c.wait()
            rdma_uk.wait()
            rdma_uv.wait()
            cm = c_mine[...]
            cp = c_peer[...]
            for g in range(H // 4):
                cs = slice(g * 4 * Dh, (g + 1) * 4 * Dh)
                k4 = (jnp.dot(cm, wuk_bf[:, cs],
                              preferred_element_type=jnp.float32)
                      + jnp.dot(cp, wuk_p[:, cs],
                                preferred_element_type=jnp.float32))
                v4 = (jnp.dot(cm, wuv_bf[:, cs],
                              preferred_element_type=jnp.float32)
                      + jnp.dot(cp, wuv_p[:, cs],
                                preferred_element_type=jnp.float32))
                for i in range(4):
                    k_scr[4 * g + i] = k4[:, i * Dh:(i + 1) * Dh]
                    v_scr[4 * g + i] = v4[:, i * Dh:(i + 1) * Dh]

        @pl.when(j >= NQ_BLK)
        def _attn():
            qr_chunk = jnp.dot(x_mine, wqr_ref[...],
                               preferred_element_type=jnp.float32)
            g = (j - NQ_BLK) * 4
            kr = kr_scr[...]
            dot_t = lambda a, b: lax.dot_general(
                a, b, (((1,), (1,)), ((), ())),
                preferred_element_type=jnp.float32)
            for i in range(4):
                q = q_scr[g + i]
                k = k_scr[g + i]
                v = v_scr[g + i]
                qr = qr_chunk[:, i * Dr:(i + 1) * Dr]
                s = (dot_t(q, k) + dot_t(qr, kr)) * scale
                p = jnp.exp(s)
                p = p / jnp.sum(p, axis=-1, keepdims=True)
                o_ref[:, i * Dh:(i + 1) * Dh] = jnp.dot(
                    p, v, preferred_element_type=jnp.float32)

    full = lambda shape: pl.BlockSpec(shape, lambda j: (0,) * len(shape))
    wq_spec = pl.BlockSpec((D, 256), lambda j: (0, jnp.minimum(j, NQ_BLK - 1)))
    wqr_spec = pl.BlockSpec(
        (D, 256), lambda j: (0, jnp.clip(j - NQ_BLK, 0, NQR_BLK - 1)))

    return pl.pallas_call(
        body,
        grid=(GRID_A,),
        in_specs=[
            full((B * S, D)),
            full((D, dc)),
            full((dc, D)),
            full((dc, D)),
            full((D, Dr)),
            wq_spec,
            wqr_spec,
        ],
        out_specs=pl.BlockSpec(
            (MB, 4 * Dh), lambda j: (0, jnp.clip(j - NQ_BLK, 0, NQR_BLK - 1))),
        out_shape=jax.ShapeDtypeStruct((MB, D), jnp.float32),
        scratch_shapes=[
            pltpu.VMEM((H, MB, Dh), jnp.float32),
            pltpu.VMEM((S, Dr), jnp.float32),
            pltpu.VMEM((H, S, Dh), jnp.float32),
            pltpu.VMEM((H, S, Dh), jnp.float32),
            pltpu.VMEM((S, dc), jnp.bfloat16),
            pltpu.VMEM((S, dc), jnp.bfloat16),
            pltpu.VMEM((dc, D), jnp.bfloat16),
            pltpu.VMEM((dc, D), jnp.bfloat16),
            pltpu.VMEM((dc, D), jnp.bfloat16),
            pltpu.VMEM((dc, D), jnp.bfloat16),
            pltpu.SemaphoreType.DMA((3,)),
            pltpu.SemaphoreType.DMA((3,)),
        ],
        compiler_params=pltpu.CompilerParams(
            collective_id=0, vmem_limit_bytes=_VMEM_LIMIT,
            dimension_semantics=("arbitrary",)),
    )(x2, Wdkv, Wuk, Wuv, Wkr, Wq, Wqr)


NO_BLK = 8


def _out_allgather(O_m, Wo):
    BN = D // NO_BLK
    SPLITS = ((0, 48), (48, 48), (96, 32))

    def body(o_ref, wo_ref, out_ref, scr, stage_c, stage_b,
             csem, bsem, send_sems, recv_sems):
        j = pl.program_id(0)
        bx = lax.axis_index("x")
        by = lax.axis_index("y")
        bz = lax.axis_index("z")
        nbrs = [(1 - bx, by, bz), (bx, 1 - by, bz), (bx, by, 1 - bz)]

        def blk(cx, cy, cz):
            return ((cy * 2 + cz) * 2 + cx) * MB

        me = blk(bx, by, bz)
        x_o = blk(1 - bx, by, bz)
        y_o = blk(bx, 1 - by, bz)
        z_o = blk(bx, by, 1 - bz)
        xy_o = blk(1 - bx, 1 - by, bz)
        xz_o = blk(1 - bx, by, 1 - bz)
        yz_o = blk(bx, 1 - by, 1 - bz)
        anti_o = blk(1 - bx, 1 - by, 1 - bz)

        def chunk_copy(slot, col_off):
            return pltpu.make_async_copy(
                stage_c.at[slot],
                out_ref.at[me // S, pl.ds(me % S, MB), pl.ds(col_off, BN)],
                csem.at[slot])

        def block_copy(slot, row_off):
            return pltpu.make_async_copy(
                stage_b.at[slot],
                out_ref.at[row_off // S, pl.ds(row_off % S, MB), :],
                bsem.at[slot])

        def xfer(link, slot, src_off, n_rows, col_off=0, n_cols=D):
            return pltpu.make_async_remote_copy(
                src_ref=scr.at[pl.ds(src_off, n_rows), pl.ds(col_off, n_cols)],
                dst_ref=scr.at[pl.ds(src_off, n_rows), pl.ds(col_off, n_cols)],
                send_sem=send_sems.at[link, slot],
                recv_sem=recv_sems.at[link, slot],
                device_id=nbrs[link], device_id_type=_MESH)

        @pl.when(j == 0)
        def _barrier():
            barrier = pltpu.get_barrier_semaphore()
            for n in nbrs:
                pl.semaphore_signal(barrier, inc=1, device_id=n,
                                    device_id_type=_MESH)
            pl.semaphore_wait(barrier, 3)

        chunk = jnp.dot(o_ref[...], wo_ref[...],
                        preferred_element_type=jnp.float32)
        slot = lax.rem(j, 2)

        @pl.when(j >= 2)
        def _reuse():
            chunk_copy(slot, (j - 2) * BN).wait()

        stage_c[slot] = chunk
        chunk_copy(slot, j * BN).start()
        scr[pl.ds(me, MB), pl.ds(j * BN, BN)] = chunk.astype(jnp.bfloat16)
        for l in range(3):
            xfer(l, j, me, MB, j * BN, BN).start()

        FWD = ((0, y_o, 1), (1, z_o, 2), (2, x_o, 0))

        def fwd_chunk(jj):
            for out_l, off, in_l in FWD:
                xfer(in_l, jj, off, MB, jj * BN, BN).wait_recv()
                xfer(out_l, NO_BLK + jj, off, MB, jj * BN, BN).start()

        @pl.when(jnp.logical_and(j >= 1, j < NO_BLK - 1))
        def _fwd():
            fwd_chunk(j - 1)

        @pl.when(j == NO_BLK - 1)
        def _tail():
            fwd_chunk(NO_BLK - 2)
            fwd_chunk(NO_BLK - 1)

            def stage_out(offs, base):
                for i, off in enumerate(offs):
                    sl = (base + i) % 2
                    stage_b[sl] = scr[pl.ds(off, MB), :].astype(jnp.float32)
                    block_copy(sl, off).start()

            def wait_out(offs, base):
                for i, off in enumerate(offs):
                    block_copy((base + i) % 2, off).wait()

            stage_out((x_o, y_o), 0)
            wait_out((x_o, y_o), 0)
            stage_out((z_o,), 0)
            for in_l, off in ((0, xy_o), (1, yz_o), (2, xz_o)):
                for jj in range(NO_BLK):
                    xfer(in_l, NO_BLK + jj, off, MB, jj * BN, BN).wait_recv()
            srcs = (yz_o, xz_o, xy_o)
            s3 = [xfer(l, 2 * NO_BLK, srcs[l] + SPLITS[l][0], SPLITS[l][1])
                  for l in range(3)]
            for r in s3:
                r.start()
            wait_out((z_o,), 0)
            stage_out((xy_o, xz_o), 1)
            wait_out((xy_o, xz_o), 1)
            stage_out((yz_o,), 1)
            for r in s3:
                r.wait()
            wait_out((yz_o,), 1)
            stage_out((anti_o,), 0)
            wait_out((anti_o,), 0)
            for l in range(3):
                for jj in range(NO_BLK):
                    xfer(l, jj, me, MB, jj * BN, BN).wait_send()
            for out_l, off, _ in FWD:
                for jj in range(NO_BLK):
                    xfer(out_l, NO_BLK + jj, off, MB,
                         jj * BN, BN).wait_send()
            chunk_copy(0, (NO_BLK - 2) * BN).wait()
            chunk_copy(1, (NO_BLK - 1) * BN).wait()

    return pl.pallas_call(
        body,
        grid=(NO_BLK,),
        in_specs=[
            pl.BlockSpec((MB, D), lambda j: (0, 0)),
            pl.BlockSpec((D, BN), lambda j: (0, j)),
        ],
        out_specs=pl.BlockSpec(memory_space=pltpu.MemorySpace.HBM),
        out_shape=jax.ShapeDtypeStruct((B, S, D), jnp.float32),
        scratch_shapes=[
            pltpu.VMEM((B * S, D), jnp.bfloat16),
            pltpu.VMEM((2, MB, D // NO_BLK), jnp.float32),
            pltpu.VMEM((2, MB, D), jnp.float32),
            pltpu.SemaphoreType.DMA((2,)),
            pltpu.SemaphoreType.DMA((2,)),
            pltpu.SemaphoreType.DMA((3, 2 * NO_BLK + 1)),
            pltpu.SemaphoreType.DMA((3, 2 * NO_BLK + 1)),
        ],
        compiler_params=pltpu.CompilerParams(
            collective_id=1, vmem_limit_bytes=_VMEM_LIMIT,
            dimension_semantics=("arbitrary",)),
    )(O_m, Wo)


def kernel(x, Wdkv, Wuk, Wuv, Wq, Wqr, Wkr, Wo):
    x2 = x.reshape(B * S, D)
    O_m = _proj_attn(x2, Wdkv, Wuk, Wuv, Wkr, Wq, Wqr)
    return _out_allgather(O_m, Wo)
